# Initial kernel scaffold; baseline (speedup 1.0000x reference)
import jax
import jax.numpy as jnp
from jax.experimental import pallas as pl


def kernel(all_timestamps, ts_w, pos_w):
    raise NotImplementedError("write your pallas kernel here")


# speedup vs baseline: 1146.7456x; 1146.7456x over previous
import functools

import jax
import jax.numpy as jnp
from jax.experimental import pallas as pl

N = 2048
RB = 128
NJ = N // 128


def _bias_kernel(ts_s_ref, ts_ref, tab_ref, pos_ref, out_ref):
    i_blk = pl.program_id(1)

    s_col = ts_s_ref[0].reshape(RB, 1)
    ts_row = ts_ref[0]

    tab = jnp.broadcast_to(tab_ref[0].reshape(1, 128), (RB, 128))

    jl = jax.lax.broadcasted_iota(jnp.int32, (RB, 128), 1)
    il = jax.lax.broadcasted_iota(jnp.int32, (RB, 128), 0)
    q = jl - il + 127
    hi = q >= 128
    qm = jnp.where(hi, q - 128, q)

    for jt in range(NJ):
        tcol = jax.lax.slice(ts_row, (0, jt * 128), (1, (jt + 1) * 128))
        diff = s_col - tcol
        mag = jnp.maximum(jnp.abs(diff), 1).astype(jnp.float32)
        bk = jnp.clip((jnp.log(mag) / 0.301).astype(jnp.int32), 0, 127)
        tsb = jnp.take_along_axis(tab, bk, axis=1, mode="promise_in_bounds")

        base = (N - 1 - 127) + 128 * (jt - i_blk)
        t0 = jnp.broadcast_to(pos_ref[0, pl.ds(base, 128)].reshape(1, 128), (RB, 128))
        t1 = jnp.broadcast_to(
            pos_ref[0, pl.ds(base + 128, 128)].reshape(1, 128), (RB, 128)
        )
        p0 = jnp.take_along_axis(t0, qm, axis=1, mode="promise_in_bounds")
        p1 = jnp.take_along_axis(t1, qm, axis=1, mode="promise_in_bounds")
        posv = jnp.where(hi, p1, p0)

        out_ref[0, :, jt * 128 : (jt + 1) * 128] = tsb + posv


@jax.jit
def kernel(all_timestamps, ts_w, pos_w):
    B = all_timestamps.shape[0]
    ts = all_timestamps.astype(jnp.int32)
    ts_sh = jnp.concatenate([ts[:, 1:], ts[:, N - 1 : N]], axis=1)
    ts_s3 = ts_sh.reshape(B * (N // RB), 1, RB)
    ts3 = ts.reshape(B, 1, N)
    tab = ts_w[:128].reshape(1, 128)
    posp = jnp.concatenate([pos_w, jnp.zeros((1,), jnp.float32)]).reshape(1, 4096)

    grid = (B, N // RB)
    out = pl.pallas_call(
        _bias_kernel,
        grid=grid,
        in_specs=[
            pl.BlockSpec((1, 1, RB), lambda b, i: (b * (N // RB) + i, 0, 0)),
            pl.BlockSpec((1, 1, N), lambda b, i: (b, 0, 0)),
            pl.BlockSpec((1, 128), lambda b, i: (0, 0)),
            pl.BlockSpec((1, 4096), lambda b, i: (0, 0)),
        ],
        out_specs=pl.BlockSpec((1, RB, N), lambda b, i: (b, i, 0)),
        out_shape=jax.ShapeDtypeStruct((B, N, N), jnp.float32),
    )(ts_s3, ts3, tab, posp)
    return out


# speedup vs baseline: 2395.1687x; 2.0887x over previous
import jax
import jax.numpy as jnp
from jax.experimental import pallas as pl
from jax.experimental.pallas import tpu as pltpu

N = 2048
RB = 128
NJ = N // 128
ND = 31


def _pos_band_kernel(pos_ref, p_ref):
    jl = jax.lax.broadcasted_iota(jnp.int32, (RB, 128), 1)
    il = jax.lax.broadcasted_iota(jnp.int32, (RB, 128), 0)
    q = jl - il + 127
    hi = q >= 128
    qm = jnp.where(hi, q - 128, q)
    for d in range(ND):
        t0 = jnp.broadcast_to(
            pos_ref[0, pl.ds(d * 128, 128)].reshape(1, 128), (RB, 128)
        )
        t1 = jnp.broadcast_to(
            pos_ref[0, pl.ds(d * 128 + 128, 128)].reshape(1, 128), (RB, 128)
        )
        p0 = jnp.take_along_axis(t0, qm, axis=1, mode="promise_in_bounds")
        p1 = jnp.take_along_axis(t1, qm, axis=1, mode="promise_in_bounds")
        p_ref[:, d * 128 : (d + 1) * 128] = jnp.where(hi, p1, p0)
    p_ref[:, ND * 128 :] = jnp.zeros((RB, 4096 - ND * 128), jnp.float32)


def _bias_kernel(ts_s_ref, ts_ref, tab_ref, p_ref, out_ref):
    i_blk = pl.program_id(1)

    s_col = ts_s_ref[0].reshape(RB, 1)
    ts_row = ts_ref[0]

    tab = jnp.broadcast_to(tab_ref[0].reshape(1, 128), (RB, 128))

    for jt in range(NJ):
        tcol = jax.lax.slice(ts_row, (0, jt * 128), (1, (jt + 1) * 128))
        diff = s_col - tcol
        mag = jnp.maximum(jnp.abs(diff.astype(jnp.float32)), 1.0)
        bk = jnp.minimum((jnp.log(mag) / 0.301).astype(jnp.int32), 127)
        tsb = jnp.take_along_axis(tab, bk, axis=1, mode="promise_in_bounds")
        posv = p_ref[:, pl.ds((15 - i_blk + jt) * 128, 128)]
        out_ref[0, :, jt * 128 : (jt + 1) * 128] = tsb + posv


@jax.jit
def kernel(all_timestamps, ts_w, pos_w):
    B = all_timestamps.shape[0]
    ts = all_timestamps.astype(jnp.int32)
    ts_sh = jnp.concatenate([ts[:, 1:], ts[:, N - 1 : N]], axis=1)
    ts_s3 = ts_sh.reshape(B * (N // RB), 1, RB)
    ts3 = ts.reshape(B, 1, N)
    tab = ts_w[:128].reshape(1, 128)
    posp = jnp.concatenate([pos_w, jnp.zeros((1,), jnp.float32)]).reshape(1, 4096)

    p_band = pl.pallas_call(
        _pos_band_kernel,
        out_shape=jax.ShapeDtypeStruct((RB, 4096), jnp.float32),
    )(posp)

    grid = (B, N // RB)
    out = pl.pallas_call(
        _bias_kernel,
        grid=grid,
        in_specs=[
            pl.BlockSpec((1, 1, RB), lambda b, i: (b * (N // RB) + i, 0, 0)),
            pl.BlockSpec((1, 1, N), lambda b, i: (b, 0, 0)),
            pl.BlockSpec((1, 128), lambda b, i: (0, 0)),
            pl.BlockSpec((RB, 4096), lambda b, i: (0, 0)),
        ],
        out_specs=pl.BlockSpec((1, RB, N), lambda b, i: (b, i, 0)),
        out_shape=jax.ShapeDtypeStruct((B, N, N), jnp.float32),
        compiler_params=pltpu.CompilerParams(
            dimension_semantics=("parallel", "parallel"),
        ),
    )(ts_s3, ts3, tab, p_band)
    return out


# speedup vs baseline: 2494.7138x; 1.0416x over previous
import jax
import jax.numpy as jnp
from jax.experimental import pallas as pl
from jax.experimental.pallas import tpu as pltpu

N = 2048
RB = 128
ND = 31


def _take(tab, idx):
    return jnp.take_along_axis(tab, idx, axis=1, mode="promise_in_bounds")


def _build_pos_band(pos_ref, p_ref):
    jl = jax.lax.broadcasted_iota(jnp.int32, (RB, 128), 1)
    il = jax.lax.broadcasted_iota(jnp.int32, (RB, 128), 0)
    q = jl - il + 127
    hi = q >= 128
    qm = jnp.where(hi, q - 128, q)
    for d in range(ND):
        t0 = jnp.broadcast_to(
            pos_ref[0, pl.ds(d * 128, 128)].reshape(1, 128), (RB, 128)
        )
        t1 = jnp.broadcast_to(
            pos_ref[0, pl.ds(d * 128 + 128, 128)].reshape(1, 128), (RB, 128)
        )
        p_ref[:, d * 128 : (d + 1) * 128] = jnp.where(hi, _take(t1, qm), _take(t0, qm))


def _bias_kernel(ts_s_ref, ts_ref, tab_ref, pos_ref, out_ref, p_ref):
    b = pl.program_id(0)
    i_blk = pl.program_id(1)

    @pl.when(jnp.logical_and(b == 0, i_blk == 0))
    def _init():
        _build_pos_band(pos_ref, p_ref)

    s_col = ts_s_ref[0].reshape(RB, 1)
    ts_row = ts_ref[0]
    tab = jnp.broadcast_to(tab_ref[...], (RB, 128))

    diff = (s_col - ts_row).astype(jnp.float32)
    mag = jnp.maximum(jnp.abs(diff), 1.0)
    bk = jnp.minimum(jnp.log(mag) / 0.301, 127.0).astype(jnp.int32)
    tsb = _take(tab, bk)
    posv = p_ref[:, pl.ds((15 - i_blk) * 128, N)]
    out_ref[0] = tsb + posv


@jax.jit
def kernel(all_timestamps, ts_w, pos_w):
    B = all_timestamps.shape[0]
    ts = all_timestamps.astype(jnp.int32)
    ts_sh = jnp.concatenate([ts[:, 1:], ts[:, N - 1 : N]], axis=1)
    ts_s3 = ts_sh.reshape(B * (N // RB), 1, RB)
    ts3 = ts.reshape(B, 1, N)
    tab = ts_w[:128].reshape(1, 128)
    posp = jnp.concatenate([pos_w, jnp.zeros((1,), jnp.float32)]).reshape(1, 4096)

    grid = (B, N // RB)
    out = pl.pallas_call(
        _bias_kernel,
        grid=grid,
        in_specs=[
            pl.BlockSpec((1, 1, RB), lambda b, i: (b * (N // RB) + i, 0, 0)),
            pl.BlockSpec((1, 1, N), lambda b, i: (b, 0, 0)),
            pl.BlockSpec((1, 128), lambda b, i: (0, 0)),
            pl.BlockSpec((1, 4096), lambda b, i: (0, 0)),
        ],
        out_specs=pl.BlockSpec((1, RB, N), lambda b, i: (b, i, 0)),
        out_shape=jax.ShapeDtypeStruct((B, N, N), jnp.float32),
        scratch_shapes=[
            pltpu.VMEM((RB, 4096), jnp.float32),
        ],
    )(ts_s3, ts3, tab, posp)
    return out


# speedup vs baseline: 2719.4483x; 1.0901x over previous
import jax
import jax.numpy as jnp
from jax.experimental import pallas as pl
from jax.experimental.pallas import tpu as pltpu

N = 2048
RB = 128
GB = 512
ND = 31


def _take(tab, idx):
    return jnp.take_along_axis(tab, idx, axis=1, mode="promise_in_bounds")


def _build_pos_band(pos_ref, p_ref):
    jl = jax.lax.broadcasted_iota(jnp.int32, (RB, 128), 1)
    il = jax.lax.broadcasted_iota(jnp.int32, (RB, 128), 0)
    q = jl - il + 127
    hi = q >= 128
    qm = jnp.where(hi, q - 128, q)
    for d in range(ND):
        t0 = jnp.broadcast_to(
            pos_ref[0, pl.ds(d * 128, 128)].reshape(1, 128), (RB, 128)
        )
        t1 = jnp.broadcast_to(
            pos_ref[0, pl.ds(d * 128 + 128, 128)].reshape(1, 128), (RB, 128)
        )
        p_ref[:, d * 128 : (d + 1) * 128] = jnp.where(hi, _take(t1, qm), _take(t0, qm))


def _bias_kernel(ts_s_ref, ts_ref, tab_ref, pos_ref, out_ref, p_ref):
    b = pl.program_id(0)
    i_blk = pl.program_id(1)

    @pl.when(jnp.logical_and(b == 0, i_blk == 0))
    def _init():
        _build_pos_band(pos_ref, p_ref)

    ts_row = ts_ref[0]
    tab = jnp.broadcast_to(tab_ref[...], (RB, 128))

    for r in range(GB // RB):
        s_col = ts_s_ref[0, 0, pl.ds(r * RB, RB)].reshape(RB, 1)
        diff = (s_col - ts_row).astype(jnp.float32)
        mag = jnp.maximum(jnp.abs(diff), 1.0)
        bk = jnp.minimum(jnp.log(mag) / 0.301, 127.0).astype(jnp.int32)
        tsb = _take(tab, bk)
        band = i_blk * (GB // RB) + r
        posv = p_ref[:, pl.ds((15 - band) * 128, N)]
        out_ref[0, r * RB : (r + 1) * RB, :] = tsb + posv


@jax.jit
def kernel(all_timestamps, ts_w, pos_w):
    B = all_timestamps.shape[0]
    ts = all_timestamps.astype(jnp.int32)
    ts_sh = jnp.concatenate([ts[:, 1:], ts[:, N - 1 : N]], axis=1)
    ts_s3 = ts_sh.reshape(B * (N // GB), 1, GB)
    ts3 = ts.reshape(B, 1, N)
    tab = ts_w[:128].reshape(1, 128)
    posp = jnp.concatenate([pos_w, jnp.zeros((1,), jnp.float32)]).reshape(1, 4096)

    grid = (B, N // GB)
    out = pl.pallas_call(
        _bias_kernel,
        grid=grid,
        in_specs=[
            pl.BlockSpec((1, 1, GB), lambda b, i: (b * (N // GB) + i, 0, 0)),
            pl.BlockSpec((1, 1, N), lambda b, i: (b, 0, 0)),
            pl.BlockSpec((1, 128), lambda b, i: (0, 0)),
            pl.BlockSpec((1, 4096), lambda b, i: (0, 0)),
        ],
        out_specs=pl.BlockSpec((1, GB, N), lambda b, i: (b, i, 0)),
        out_shape=jax.ShapeDtypeStruct((B, N, N), jnp.float32),
        scratch_shapes=[
            pltpu.VMEM((RB, 4096), jnp.float32),
        ],
    )(ts_s3, ts3, tab, posp)
    return out
